# baseline (speedup 1.0000x reference)
import jax
import jax.numpy as jnp
from jax import lax
from jax.experimental import pallas as pl
from jax.experimental.pallas import tpu as pltpu
from jax.experimental.pallas import tpu_sc as plsc

_B = 128
_V = 100000
_H = 200
_HP = 208
_NW = 32
_RPW = _B // _NW
_W = 10000
_NC = _V // _W
_STEPS = _W // 16


def _sc_body(lg_hbm, gm_hbm, ids_hbm, pt_hbm, out_hbm,
             lg_v, gm_v, ids_v, pt_v, out_v):
    c = lax.axis_index("c")
    s = lax.axis_index("s")
    wid = s * 2 + c
    lane = lax.broadcasted_iota(jnp.int32, (16,), 0)

    def do_row(rloc, row_carry):
        row = wid * _RPW + rloc
        pltpu.sync_copy(ids_hbm.at[pl.ds(row * _HP, _HP)], ids_v)
        pltpu.sync_copy(pt_hbm.at[pl.ds(row * 32, 32)], pt_v)
        p16 = pt_v[pl.ds(0, 16)]
        t16 = pt_v[pl.ds(16, 16)]
        greedy = t16 < 1e-5
        teff = jnp.where(greedy, jnp.float32(1.0), t16)
        gsel = jnp.where(greedy, jnp.float32(0.0), jnp.float32(1.0))

        def do_chunk(ci, carry):
            bestv, besti = carry
            off = ci * _W
            pltpu.sync_copy(lg_hbm.at[pl.ds(row * _V + off, _W)], lg_v)
            pltpu.sync_copy(gm_hbm.at[pl.ds(row * _V + off, _W)], gm_v)

            fixes = []
            for j in range(_HP // 16):
                idv = ids_v[pl.ds(j * 16, 16)]
                m = (idv >= off) & (idv < off + _W)
                loc = jnp.where(m, idv - off, 0)
                g = plsc.load_gather(lg_v, [loc], mask=m)
                fixes.append((loc, g - p16, m))
            for loc, val, m in fixes:
                plsc.store_scatter(lg_v, [loc], val, mask=m)

            def step(si, sc):
                bv, bi = sc
                base = off + si * 16
                lg16 = lg_v[pl.ds(si * 16, 16)]
                gm16 = gm_v[pl.ds(si * 16, 16)]
                val = lg16 / teff + gm16 * gsel
                idx = base + lane
                upd = val > bv
                return (jnp.where(upd, val, bv), jnp.where(upd, idx, bi))

            return lax.fori_loop(0, _STEPS, step, (bestv, besti), unroll=4)

        bestv0 = jnp.full((16,), -3.0e38, jnp.float32)
        besti0 = jnp.zeros((16,), jnp.int32)
        bestv, besti = lax.fori_loop(0, _NC, do_chunk, (bestv0, besti0))

        m = jnp.max(bestv)
        cand = jnp.where(bestv == m, besti, jnp.int32(2**30))
        mi = jnp.min(cand)
        out_v[pl.ds(rloc * 16, 16)] = jnp.full((16,), mi, jnp.int32)
        return row_carry

    lax.fori_loop(0, _RPW, do_row, 0)
    pltpu.sync_copy(out_v, out_hbm.at[pl.ds(wid * 64, 64)])


_GUMBEL = None


def _gumbel():
    global _GUMBEL
    if _GUMBEL is None:
        def build():
            u = jax.random.uniform(jax.random.key(42), (_B, _V),
                                   dtype=jnp.float32, minval=1e-10,
                                   maxval=1.0)
            return (-jnp.log(-jnp.log(u))).reshape(-1)
        _GUMBEL = jax.block_until_ready(jax.jit(build)())
    return _GUMBEL


def kernel(logits_next, presence_penalties, temperatures, token_ids):
    gm = _gumbel()
    lg = logits_next.reshape(-1)
    ids = jnp.pad(token_ids.astype(jnp.int32), ((0, 0), (0, _HP - _H)),
                  constant_values=_V).reshape(-1)
    pt = jnp.broadcast_to(
        jnp.stack([presence_penalties, temperatures], axis=1)[:, :, None],
        (_B, 2, 16)).astype(jnp.float32).reshape(-1)

    mesh = plsc.VectorSubcoreMesh(core_axis_name="c", subcore_axis_name="s",
                                  num_cores=2, num_subcores=16)
    run = pl.kernel(
        _sc_body,
        out_type=jax.ShapeDtypeStruct((_NW * 64,), jnp.int32),
        mesh=mesh,
        scratch_types=[
            pltpu.VMEM((_W,), jnp.float32),
            pltpu.VMEM((_W,), jnp.float32),
            pltpu.VMEM((_HP,), jnp.int32),
            pltpu.VMEM((32,), jnp.float32),
            pltpu.VMEM((4 * 16,), jnp.int32),
        ],
        compiler_params=pltpu.CompilerParams(needs_layout_passes=False),
    )
    out = run(lg, gm, ids, pt)
    return out.reshape(_NW, _RPW, 16)[:, :, 0].reshape(_B)


# speedup vs baseline: 2.0509x; 2.0509x over previous
import jax
import jax.numpy as jnp
from jax import lax
from jax.experimental import pallas as pl
from jax.experimental.pallas import tpu as pltpu
from jax.experimental.pallas import tpu_sc as plsc

_B = 128
_V = 100000
_H = 200
_HP = 208
_NG = 16
_TC_ALL = 782
_TPH = 391
_T = 17
_NCH = _TPH // _T
_NEG = -3.0e38


def _sc_body(lg_hbm, gm_hbm, ids_hbm, pt_hbm, outv_hbm, outi_hbm,
             lg_v, gm_v, ids_v, pt_v, ov_v, oi_v):
    c = lax.axis_index("c")
    s = lax.axis_index("s")
    wid = s * 2 + c
    g = wid >> 1
    half = wid & 1
    lane = lax.broadcasted_iota(jnp.int32, (16,), 0)

    pltpu.sync_copy(ids_hbm.at[pl.ds(g * 8 * _HP, 8 * _HP)], ids_v)
    pltpu.sync_copy(pt_hbm.at[pl.ds(g * 8 * 32, 8 * 32)], pt_v)

    cb0 = half * _TPH

    rowp = []
    for r8 in range(8):
        p16 = pt_v[pl.ds(r8 * 32, 16)]
        t16 = pt_v[pl.ds(r8 * 32 + 16, 16)]
        greedy = t16 < 1e-5
        teff = jnp.where(greedy, jnp.float32(1.0), t16)
        gsel = jnp.where(greedy, jnp.float32(0.0), jnp.float32(1.0))
        rowp.append((p16, teff, gsel))

    def do_chunk(ci, carry):
        col0 = (cb0 + ci * _T) * 128
        pltpu.sync_copy(lg_hbm.at[g, pl.ds(0, 8), pl.ds(col0, _T * 128)],
                        lg_v)
        pltpu.sync_copy(gm_hbm.at[g, pl.ds(0, 8), pl.ds(col0, _T * 128)],
                        gm_v)

        out = []
        for r8 in range(8):
            p16, teff, gsel = rowp[r8]
            bestv, besti = carry[2 * r8], carry[2 * r8 + 1]

            r16 = jnp.full((16,), r8, jnp.int32)
            fixes = []
            for j in range(_HP // 16):
                idv = ids_v[pl.ds(r8 * _HP + j * 16, 16)]
                m = (idv >= col0) & (idv < col0 + _T * 128)
                loc = jnp.where(m, idv - col0, 0)
                val = plsc.load_gather(lg_v, [r16, loc], mask=m)
                fixes.append((loc, val - p16, m))
            for loc, val, m in fixes:
                plsc.store_scatter(lg_v, [r16, loc], val, mask=m)

            def step(k, sc, r8=r8, p16=p16, teff=teff, gsel=gsel,
                     col0=col0):
                bv, bi = sc
                lg16 = lg_v[r8, pl.ds(k * 16, 16)]
                gm16 = gm_v[r8, pl.ds(k * 16, 16)]
                idx = col0 + k * 16 + lane
                val = lg16 / teff + gm16 * gsel
                val = jnp.where(idx < _V, val, jnp.float32(_NEG))
                upd = val > bv
                return (jnp.where(upd, val, bv), jnp.where(upd, idx, bi))

            bestv, besti = lax.fori_loop(0, _T * 8, step, (bestv, besti),
                                         unroll=4)
            out.extend([bestv, besti])
        return tuple(out)

    init = []
    for _ in range(8):
        init.extend([jnp.full((16,), _NEG, jnp.float32),
                     jnp.zeros((16,), jnp.int32)])
    final = lax.fori_loop(0, _NCH, do_chunk, tuple(init))

    for r8 in range(8):
        bestv, besti = final[2 * r8], final[2 * r8 + 1]
        m = jnp.max(bestv)
        cand = jnp.where(bestv == m, besti, jnp.int32(2**30))
        mi = jnp.min(cand)
        ov_v[pl.ds(r8 * 16, 16)] = jnp.full((16,), m, jnp.float32)
        oi_v[pl.ds(r8 * 16, 16)] = jnp.full((16,), mi, jnp.int32)

    pltpu.sync_copy(ov_v, outv_hbm.at[pl.ds(wid * 128, 128)])
    pltpu.sync_copy(oi_v, outi_hbm.at[pl.ds(wid * 128, 128)])


_GUMBEL = None


def _gumbel():
    global _GUMBEL
    if _GUMBEL is None:
        def build():
            u = jax.random.uniform(jax.random.key(42), (_B, _V),
                                   dtype=jnp.float32, minval=1e-10,
                                   maxval=1.0)
            return (-jnp.log(-jnp.log(u))).reshape(_NG, 8, _V)
        _GUMBEL = jax.block_until_ready(jax.jit(build)())
    return _GUMBEL


def kernel(logits_next, presence_penalties, temperatures, token_ids):
    gm = _gumbel()
    lg = logits_next.reshape(_NG, 8, _V)
    ids = jnp.pad(token_ids.astype(jnp.int32), ((0, 0), (0, _HP - _H)),
                  constant_values=_V).reshape(-1)
    pt = jnp.broadcast_to(
        jnp.stack([presence_penalties, temperatures], axis=1)[:, :, None],
        (_B, 2, 16)).astype(jnp.float32).reshape(-1)

    mesh = plsc.VectorSubcoreMesh(core_axis_name="c", subcore_axis_name="s",
                                  num_cores=2, num_subcores=16)
    run = pl.kernel(
        _sc_body,
        out_type=(jax.ShapeDtypeStruct((32 * 128,), jnp.float32),
                  jax.ShapeDtypeStruct((32 * 128,), jnp.int32)),
        mesh=mesh,
        scratch_types=[
            pltpu.VMEM((8, _T * 128), jnp.float32),
            pltpu.VMEM((8, _T * 128), jnp.float32),
            pltpu.VMEM((8 * _HP,), jnp.int32),
            pltpu.VMEM((8 * 32,), jnp.float32),
            pltpu.VMEM((128,), jnp.float32),
            pltpu.VMEM((128,), jnp.int32),
        ],
        compiler_params=pltpu.CompilerParams(needs_layout_passes=False,
                                             use_tc_tiling_on_sc=True),
    )
    vals, idxs = run(lg, gm, ids, pt)
    v = vals.reshape(32, 8, 16)[:, :, 0]
    i = idxs.reshape(32, 8, 16)[:, :, 0]
    v0, v1 = v[0::2], v[1::2]
    i0, i1 = i[0::2], i[1::2]
    out = jnp.where(v1 > v0, i1, i0)
    return out.reshape(_B)


# speedup vs baseline: 2.0516x; 1.0003x over previous
import jax
import jax.numpy as jnp
from jax import lax
from jax.experimental import pallas as pl
from jax.experimental.pallas import tpu as pltpu
from jax.experimental.pallas import tpu_sc as plsc

_B = 128
_V = 100000
_H = 200
_HP = 208
_NG = 16
_TC_ALL = 782
_TPH = 391
_T = 17
_NCH = _TPH // _T
_NEG = -3.0e38


def _sc_body(lg_hbm, gm_hbm, ids_hbm, pt_hbm, outv_hbm, outi_hbm,
             lg_v, gm_v, ids_v, pt_v, ov_v, oi_v):
    c = lax.axis_index("c")
    s = lax.axis_index("s")
    wid = s * 2 + c
    g = wid >> 1
    half = wid & 1
    lane = lax.broadcasted_iota(jnp.int32, (16,), 0)

    pltpu.sync_copy(ids_hbm.at[pl.ds(g * 8 * _HP, 8 * _HP)], ids_v)
    pltpu.sync_copy(pt_hbm.at[pl.ds(g * 8 * 32, 8 * 32)], pt_v)

    cb0 = half * _TPH

    rowp = []
    for r8 in range(8):
        p16 = pt_v[pl.ds(r8 * 32, 16)]
        t16 = pt_v[pl.ds(r8 * 32 + 16, 16)]
        greedy = t16 < 1e-5
        teff = jnp.where(greedy, jnp.float32(1.0), t16)
        gsel = jnp.where(greedy, jnp.float32(0.0), jnp.float32(1.0))
        rowp.append((p16, teff, gsel))

    def do_chunk(ci, carry):
        col0 = (cb0 + ci * _T) * 128
        pltpu.sync_copy(lg_hbm.at[g, pl.ds(0, 8), pl.ds(col0, _T * 128)],
                        lg_v)
        pltpu.sync_copy(gm_hbm.at[g, pl.ds(0, 8), pl.ds(col0, _T * 128)],
                        gm_v)

        out = []
        for r8 in range(8):
            p16, teff, gsel = rowp[r8]
            bestv, besti = carry[2 * r8], carry[2 * r8 + 1]

            r16 = jnp.full((16,), r8, jnp.int32)
            fixes = []
            for j in range(_HP // 16):
                idv = ids_v[pl.ds(r8 * _HP + j * 16, 16)]
                m = (idv >= col0) & (idv < col0 + _T * 128)
                loc = jnp.where(m, idv - col0, 0)
                val = plsc.load_gather(lg_v, [r16, loc], mask=m)
                fixes.append((loc, val - p16, m))
            for loc, val, m in fixes:
                plsc.store_scatter(lg_v, [r16, loc], val, mask=m)

            def step(k, sc, r8=r8, p16=p16, teff=teff, gsel=gsel,
                     col0=col0):
                bv, bi = sc
                lg16 = lg_v[r8, pl.ds(k * 16, 16)]
                gm16 = gm_v[r8, pl.ds(k * 16, 16)]
                idx = col0 + k * 16 + lane
                val = lg16 / teff + gm16 * gsel
                val = jnp.where(idx < _V, val, jnp.float32(_NEG))
                upd = val > bv
                return (jnp.where(upd, val, bv), jnp.where(upd, idx, bi))

            bestv, besti = lax.fori_loop(0, _T * 8, step, (bestv, besti),
                                         unroll=4)
            out.extend([bestv, besti])
        return tuple(out)

    init = []
    for _ in range(8):
        init.extend([jnp.full((16,), _NEG, jnp.float32),
                     jnp.zeros((16,), jnp.int32)])
    final = lax.fori_loop(0, _NCH, do_chunk, tuple(init))

    for r8 in range(8):
        bestv, besti = final[2 * r8], final[2 * r8 + 1]
        m = jnp.max(bestv)
        cand = jnp.where(bestv == m, besti, jnp.int32(2**30))
        mi = jnp.min(cand)
        ov_v[pl.ds(r8 * 16, 16)] = jnp.full((16,), m, jnp.float32)
        oi_v[pl.ds(r8 * 16, 16)] = jnp.full((16,), mi, jnp.int32)

    pltpu.sync_copy(ov_v, outv_hbm.at[pl.ds(wid * 128, 128)])
    pltpu.sync_copy(oi_v, outi_hbm.at[pl.ds(wid * 128, 128)])


_GUMBEL = None


def _gumbel():
    global _GUMBEL
    if _GUMBEL is None:
        def build():
            u = jax.random.uniform(jax.random.key(42), (_B, _V),
                                   dtype=jnp.float32, minval=1e-10,
                                   maxval=1.0)
            return (-jnp.log(-jnp.log(u))).reshape(_NG, 8, _V)
        _GUMBEL = jax.block_until_ready(jax.jit(build)())
    return _GUMBEL


def kernel(logits_next, presence_penalties, temperatures, token_ids):
    gm = _gumbel()
    lgT = lax.optimization_barrier(jnp.swapaxes(logits_next, 0, 1))
    lg = jnp.swapaxes(lgT, 0, 1).reshape(_NG, 8, _V)
    ids = jnp.pad(token_ids.astype(jnp.int32), ((0, 0), (0, _HP - _H)),
                  constant_values=_V).reshape(-1)
    pt = jnp.broadcast_to(
        jnp.stack([presence_penalties, temperatures], axis=1)[:, :, None],
        (_B, 2, 16)).astype(jnp.float32).reshape(-1)

    mesh = plsc.VectorSubcoreMesh(core_axis_name="c", subcore_axis_name="s",
                                  num_cores=2, num_subcores=16)
    run = pl.kernel(
        _sc_body,
        out_type=(jax.ShapeDtypeStruct((32 * 128,), jnp.float32),
                  jax.ShapeDtypeStruct((32 * 128,), jnp.int32)),
        mesh=mesh,
        scratch_types=[
            pltpu.VMEM((8, _T * 128), jnp.float32),
            pltpu.VMEM((8, _T * 128), jnp.float32),
            pltpu.VMEM((8 * _HP,), jnp.int32),
            pltpu.VMEM((8 * 32,), jnp.float32),
            pltpu.VMEM((128,), jnp.float32),
            pltpu.VMEM((128,), jnp.int32),
        ],
        compiler_params=pltpu.CompilerParams(needs_layout_passes=False,
                                             use_tc_tiling_on_sc=True),
    )
    vals, idxs = run(lg, gm, ids, pt)
    v = vals.reshape(32, 8, 16)[:, :, 0]
    i = idxs.reshape(32, 8, 16)[:, :, 0]
    v0, v1 = v[0::2], v[1::2]
    i0, i1 = i[0::2], i[1::2]
    out = jnp.where(v1 > v0, i1, i0)
    return out.reshape(_B)
